# baseline (speedup 1.0000x reference)
import jax
import jax.numpy as jnp
from jax.experimental import pallas as pl

_N_CLASSES = 1000
_W = 500
_BB = 8


def _onehot_block(tt_ref, out_ref):
    bb, J, W = out_ref.shape
    S = 50
    tt = tt_ref[...]
    j = jax.lax.broadcasted_iota(jnp.int32, (bb, J, W), 1)
    m = jax.lax.broadcasted_iota(jnp.int32, (bb, J, W), 2)
    cls = j * (W // S) + m // S
    out_ref[...] = (cls == tt[:, None, :]).astype(jnp.float32)


def kernel(t) -> jnp.ndarray:
    B, S = t.shape
    C = _N_CLASSES
    W = _W
    J = (C * S) // W
    bb = _BB
    tt = jnp.tile(t.astype(jnp.int32), (1, W // S))
    out2 = pl.pallas_call(
        _onehot_block,
        grid=(B // bb,),
        in_specs=[pl.BlockSpec((bb, W), lambda i: (i, 0))],
        out_specs=pl.BlockSpec((bb, J, W), lambda i: (i, 0, 0)),
        out_shape=jax.ShapeDtypeStruct((B, J, W), jnp.float32),
    )(tt)
    return out2.reshape(B, C, S)


# speedup vs baseline: 2.2921x; 2.2921x over previous
import jax
import jax.numpy as jnp
from jax.experimental import pallas as pl
from jax.experimental.pallas import tpu as pltpu

_N_CLASSES = 1000
_BB = 8
_NBUF = 8


def _onehot_block(t_ref, out_ref, slabs, sems):
    i = pl.program_id(0)
    n = pl.num_programs(0)
    bb = _BB
    slot = jax.lax.rem(i, _NBUF)

    @pl.when(i >= _NBUF)
    def _wait_prev():
        pltpu.make_async_copy(
            slabs.at[slot], out_ref.at[pl.ds(i * bb, bb)], sems.at[slot]
        ).wait()

    t = t_ref[pl.ds(i * bb, bb), :]
    c = jax.lax.broadcasted_iota(jnp.int32, slabs.shape[1:], 1)
    slabs[slot] = (c == t[:, None, :]).astype(jnp.float32)

    pltpu.make_async_copy(
        slabs.at[slot], out_ref.at[pl.ds(i * bb, bb)], sems.at[slot]
    ).start()

    @pl.when(i == n - 1)
    def _drain():
        for k in range(_NBUF):
            pltpu.make_async_copy(
                slabs.at[k], out_ref.at[pl.ds(0, bb)], sems.at[k]
            ).wait()


def kernel(t) -> jnp.ndarray:
    B, S = t.shape
    C = _N_CLASSES
    bb = _BB
    return pl.pallas_call(
        _onehot_block,
        grid=(B // bb,),
        in_specs=[pl.BlockSpec(memory_space=pltpu.VMEM)],
        out_specs=pl.BlockSpec(memory_space=pl.ANY),
        out_shape=jax.ShapeDtypeStruct((B, C, S), jnp.float32),
        scratch_shapes=[
            pltpu.VMEM((_NBUF, bb, C, S), jnp.float32),
            pltpu.SemaphoreType.DMA((_NBUF,)),
        ],
    )(t.astype(jnp.int32))


# speedup vs baseline: 18.6756x; 8.1480x over previous
import jax
import jax.numpy as jnp
from jax.experimental import pallas as pl
from jax.experimental.pallas import tpu as pltpu

_N_CLASSES = 1000
_NBUF = 8


def _onehot_block(t_ref, out_ref, slabs, sems):
    i = pl.program_id(0)
    n = pl.num_programs(0)
    slot = jax.lax.rem(i, _NBUF)

    @pl.when(i >= _NBUF)
    def _wait_prev():
        pltpu.make_async_copy(
            slabs.at[slot], out_ref.at[pl.ds(i, 1)], sems.at[slot]
        ).wait()

    t_row = t_ref[...]
    c = jax.lax.broadcasted_iota(jnp.int32, slabs.shape[1:], 1)
    slabs[slot] = (c == t_row).astype(jnp.float32)

    pltpu.make_async_copy(
        slabs.at[slot], out_ref.at[pl.ds(i, 1)], sems.at[slot]
    ).start()

    @pl.when(i == n - 1)
    def _drain():
        for k in range(_NBUF):
            pltpu.make_async_copy(
                slabs.at[k], out_ref.at[pl.ds(0, 1)], sems.at[k]
            ).wait()


def kernel(t) -> jnp.ndarray:
    B, S = t.shape
    C = _N_CLASSES
    tt = t.astype(jnp.int32).T.reshape(S, 1, B)
    out_t = pl.pallas_call(
        _onehot_block,
        grid=(S,),
        in_specs=[pl.BlockSpec((1, 1, B), lambda i: (i, 0, 0))],
        out_specs=pl.BlockSpec(memory_space=pl.ANY),
        out_shape=jax.ShapeDtypeStruct((S, C, B), jnp.float32),
        scratch_shapes=[
            pltpu.VMEM((_NBUF, 1, C, B), jnp.float32),
            pltpu.SemaphoreType.DMA((_NBUF,)),
        ],
    )(tt)
    return jnp.transpose(out_t, (2, 1, 0))


# speedup vs baseline: 18.6813x; 1.0003x over previous
import jax
import jax.numpy as jnp
from jax.experimental import pallas as pl
from jax.experimental.pallas import tpu as pltpu

_N_CLASSES = 1000
_NBUF = 12


def _onehot_block(t_ref, out_ref, slabs, sems):
    i = pl.program_id(0)
    n = pl.num_programs(0)
    slot = jax.lax.rem(i, _NBUF)

    @pl.when(i >= _NBUF)
    def _wait_prev():
        pltpu.make_async_copy(
            slabs.at[slot], out_ref.at[pl.ds(i, 1)], sems.at[slot]
        ).wait()

    t_row = t_ref[...]
    c = jax.lax.broadcasted_iota(jnp.int32, slabs.shape[1:], 1)
    slabs[slot] = (c == t_row).astype(jnp.float32)

    pltpu.make_async_copy(
        slabs.at[slot], out_ref.at[pl.ds(i, 1)], sems.at[slot]
    ).start()

    @pl.when(i == n - 1)
    def _drain():
        for k in range(_NBUF):
            pltpu.make_async_copy(
                slabs.at[k], out_ref.at[pl.ds(0, 1)], sems.at[k]
            ).wait()


def kernel(t) -> jnp.ndarray:
    B, S = t.shape
    C = _N_CLASSES
    tt = t.astype(jnp.int32).T.reshape(S, 1, B)
    out_t = pl.pallas_call(
        _onehot_block,
        grid=(S,),
        in_specs=[pl.BlockSpec((1, 1, B), lambda i: (i, 0, 0))],
        out_specs=pl.BlockSpec(memory_space=pl.ANY),
        out_shape=jax.ShapeDtypeStruct((S, C, B), jnp.float32),
        scratch_shapes=[
            pltpu.VMEM((_NBUF, 1, C, B), jnp.float32),
            pltpu.SemaphoreType.DMA((_NBUF,)),
        ],
    )(tt)
    return jnp.transpose(out_t, (2, 1, 0))
